# baseline (speedup 1.0000x reference)
import jax
import jax.numpy as jnp
from jax import lax
from jax.experimental import pallas as pl
from jax.experimental.pallas import tpu as pltpu
from jax.experimental.pallas import tpu_sc as plsc

DIM = 64
SCALE = 8.0
LANES = 16

NUM_CORES = 2
NUM_SUBCORES = 16
NUM_WORKERS = NUM_CORES * NUM_SUBCORES

XROWS_PER_CHUNK = 2


def _make_body(n_xrows: int, xrow_len: int):
    rows_per_w = n_xrows // NUM_WORKERS
    idx_per_w = rows_per_w * xrow_len
    chunk_idx = XROWS_PER_CHUNK * xrow_len
    n_chunks = rows_per_w // XROWS_PER_CHUNK

    def body(x_hbm, table_hbm, out_hbm, idx_v, gbuf0, gbuf1, obuf0, obuf1,
             gsem0, gsem1, wsem0, wsem1):
        gbufs = (gbuf0, gbuf1)
        obufs = (obuf0, obuf1)
        gsems = (gsem0, gsem1)
        wsems = (wsem0, wsem1)
        wid = lax.axis_index("s") * NUM_CORES + lax.axis_index("c")
        base = wid * idx_per_w
        xrow0 = wid * rows_per_w
        pltpu.sync_copy(x_hbm.at[pl.ds(base, idx_per_w)], idx_v)

        def fire_gather(g, b):
            pltpu.async_copy(
                table_hbm.at[idx_v.at[pl.ds(g * chunk_idx, chunk_idx)]],
                gbufs[b], gsems[b])

        def wait_gather(b):
            pltpu.make_async_copy(
                table_hbm.at[idx_v.at[pl.ds(0, chunk_idx)]],
                gbufs[b], gsems[b]).wait()

        def fire_write(g, b):
            pltpu.async_copy(
                obufs[b],
                out_hbm.at[pl.ds(xrow0 + g * XROWS_PER_CHUNK, XROWS_PER_CHUNK)],
                wsems[b])

        def wait_write(b):
            pltpu.make_async_copy(
                obufs[b], out_hbm.at[pl.ds(0, XROWS_PER_CHUNK)],
                wsems[b]).wait()

        def scale(b):
            for a in range(XROWS_PER_CHUNK):
                @pl.loop(0, xrow_len, unroll=4)
                def _s(r, b=b, a=a):
                    for k in range(DIM // LANES):
                        cs = pl.ds(k * LANES, LANES)
                        obufs[b][a, r, cs] = gbufs[b][a * xrow_len + r, cs] * SCALE

        fire_gather(0, 0)
        fire_gather(1, 1)
        for g in (0, 1):
            b = g
            wait_gather(b)
            scale(b)
            fire_write(g, b)
            fire_gather(g + 2, b)

        @pl.loop(2, n_chunks - 2, step=2)
        def _mid(gg):
            for b in range(2):
                g = gg + b
                wait_gather(b)
                wait_write(b)
                scale(b)
                fire_write(g, b)
                fire_gather(g + 2, b)

        for g in (n_chunks - 2, n_chunks - 1):
            b = g % 2
            wait_gather(b)
            wait_write(b)
            scale(b)
            fire_write(g, b)
        wait_write(0)
        wait_write(1)

    return body


def kernel(x, table):
    n_xrows, xrow_len = x.shape
    xf = x.reshape(-1)
    chunk_idx = XROWS_PER_CHUNK * xrow_len

    mesh = plsc.VectorSubcoreMesh(core_axis_name="c", subcore_axis_name="s")
    out = pl.kernel(
        _make_body(n_xrows, xrow_len),
        out_type=jax.ShapeDtypeStruct((n_xrows, xrow_len, DIM), jnp.float32),
        mesh=mesh,
        compiler_params=pltpu.CompilerParams(use_tc_tiling_on_sc=False),
        scratch_types=(
            [pltpu.VMEM((xf.size // NUM_WORKERS,), jnp.int32)]
            + [pltpu.VMEM((chunk_idx, DIM), jnp.float32)] * 2
            + [pltpu.VMEM((XROWS_PER_CHUNK, xrow_len, DIM), jnp.float32)] * 2
            + [pltpu.SemaphoreType.DMA] * 4
        ),
    )(xf, table)
    return out
